# Initial kernel scaffold; baseline (speedup 1.0000x reference)
import jax
import jax.numpy as jnp
from jax.experimental import pallas as pl


def kernel(feats, segment_ids, W, b):
    raise NotImplementedError("write your pallas kernel here")


# speedup vs baseline: 2.4177x; 2.4177x over previous
import jax
import jax.numpy as jnp
from jax.experimental import pallas as pl

N_NODES = 50000
IN_FEATS = 512
NUM_GRAPHS = 256
BLOCK = 2000
NUM_BLOCKS = N_NODES // BLOCK


def _body(f_ref, s_ref, w_ref, b_ref, hg_ref, aw_ref):
    i = pl.program_id(0)
    f = f_ref[...]
    aw = jax.lax.dot_general(
        f, w_ref[...], (((1,), (0,)), ((), ())),
        preferred_element_type=jnp.float32,
        precision=jax.lax.Precision.HIGHEST,
    ) + b_ref[0, 0]
    aw_ref[...] = aw
    w = jax.nn.sigmoid(aw)
    s = s_ref[0, 0, :]
    cols = jax.lax.broadcasted_iota(jnp.int32, (BLOCK, NUM_GRAPHS), 1)
    onehot = jnp.where(s[:, None] == cols, w, 0.0)
    partial = jax.lax.dot_general(
        onehot, f, (((0,), (0,)), ((), ())),
        preferred_element_type=jnp.float32,
        precision=jax.lax.Precision.HIGHEST,
    )

    @pl.when(i == 0)
    def _():
        hg_ref[...] = partial

    @pl.when(i > 0)
    def _():
        hg_ref[...] += partial


def kernel(feats, segment_ids, W, b):
    s3 = segment_ids.astype(jnp.int32).reshape(NUM_BLOCKS, 1, BLOCK)
    b2 = b.reshape(1, 1).astype(jnp.float32)
    hg, aw = pl.pallas_call(
        _body,
        grid=(NUM_BLOCKS,),
        in_specs=[
            pl.BlockSpec((BLOCK, IN_FEATS), lambda i: (i, 0)),
            pl.BlockSpec((1, 1, BLOCK), lambda i: (i, 0, 0)),
            pl.BlockSpec((IN_FEATS, 1), lambda i: (0, 0)),
            pl.BlockSpec((1, 1), lambda i: (0, 0)),
        ],
        out_specs=[
            pl.BlockSpec((NUM_GRAPHS, IN_FEATS), lambda i: (0, 0)),
            pl.BlockSpec((BLOCK, 1), lambda i: (i, 0)),
        ],
        out_shape=[
            jax.ShapeDtypeStruct((NUM_GRAPHS, IN_FEATS), jnp.float32),
            jax.ShapeDtypeStruct((N_NODES, 1), jnp.float32),
        ],
    )(feats, s3, W, b2)
    return (hg, aw)


# speedup vs baseline: 10.1666x; 4.2051x over previous
import jax
import jax.numpy as jnp
from jax.experimental import pallas as pl

N_NODES = 50000
IN_FEATS = 512
NUM_GRAPHS = 256
BLOCK = 2000
NUM_BLOCKS = N_NODES // BLOCK


def _body(f_ref, s_ref, w_ref, b_ref, hg_ref, aw_ref):
    i = pl.program_id(0)
    f = f_ref[...]
    aw = jax.lax.dot_general(
        f, w_ref[...], (((1,), (0,)), ((), ())),
        preferred_element_type=jnp.float32,
    ) + b_ref[0, 0]
    aw_ref[...] = aw
    w = jax.nn.sigmoid(aw)
    wf = f * w
    s_row = s_ref[0, :, :]
    rows = jax.lax.broadcasted_iota(jnp.int32, (NUM_GRAPHS, BLOCK), 0)
    onehot_t = jnp.where(s_row == rows, 1.0, 0.0)
    partial = jax.lax.dot_general(
        onehot_t, wf, (((1,), (0,)), ((), ())),
        preferred_element_type=jnp.float32,
    )

    @pl.when(i == 0)
    def _():
        hg_ref[...] = partial

    @pl.when(i > 0)
    def _():
        hg_ref[...] += partial


def kernel(feats, segment_ids, W, b):
    s3 = segment_ids.astype(jnp.int32).reshape(NUM_BLOCKS, 1, BLOCK)
    b2 = b.reshape(1, 1).astype(jnp.float32)
    hg, aw = pl.pallas_call(
        _body,
        grid=(NUM_BLOCKS,),
        in_specs=[
            pl.BlockSpec((BLOCK, IN_FEATS), lambda i: (i, 0)),
            pl.BlockSpec((1, 1, BLOCK), lambda i: (i, 0, 0)),
            pl.BlockSpec((IN_FEATS, 1), lambda i: (0, 0)),
            pl.BlockSpec((1, 1), lambda i: (0, 0)),
        ],
        out_specs=[
            pl.BlockSpec((NUM_GRAPHS, IN_FEATS), lambda i: (0, 0)),
            pl.BlockSpec((BLOCK, 1), lambda i: (i, 0)),
        ],
        out_shape=[
            jax.ShapeDtypeStruct((NUM_GRAPHS, IN_FEATS), jnp.float32),
            jax.ShapeDtypeStruct((N_NODES, 1), jnp.float32),
        ],
    )(feats, s3, W, b2)
    return (hg, aw)
